# baseline (speedup 1.0000x reference)
import jax
import jax.numpy as jnp
from jax import lax
from jax.experimental import pallas as pl
from jax.experimental.pallas import tpu as pltpu, tpu_sc as plsc

_N, _C, _H_IN, _W_IN = 20, 16, 24, 30
_H_OUT, _W_OUT = 49, 61
_P = _N * _C
_S_IN = _H_IN * _W_IN
_NVEC = _S_IN // 16
_NW = 32
_PLANES_PER_W = _P // _NW
_DIV_MUL, _DIV_SHIFT = 4298, 18


def _split_hw(iv):
    hv = lax.shift_right_logical(iv * _DIV_MUL, _DIV_SHIFT)
    wv = iv - hv * _W_OUT
    return hv, wv


def _unpool_body(x_hbm, idx_hbm, out_hbm,
                 idx_v0, idx_v1, idx_v2, idx_v3, idx_v4, idx_v5, idx_v6,
                 val_v0, val_v1, val_v2, val_v3,
                 out_v0, out_v1, out_v2, in_sems, out_sems):
    c = lax.axis_index("c")
    s = lax.axis_index("s")
    wid = s * 2 + c

    idx_bufs = [idx_v0, idx_v1, idx_v2, idx_v3, idx_v4, idx_v5, idx_v6]
    val_bufs = [val_v0, val_v1, val_v2, val_v3]
    out_bufs = [out_v0, out_v1, out_v2]

    zeros16 = jnp.zeros((16,), jnp.float32)
    lanes = lax.iota(jnp.int32, 16)

    for ob in out_bufs:
        for r in range(_H_OUT):
            hv = jnp.full((16,), r, jnp.int32)
            for cb in range(0, _W_OUT, 16):
                wv = lanes + cb
                if cb + 16 <= _W_OUT:
                    plsc.store_scatter(ob, [hv, wv], zeros16)
                else:
                    plsc.store_scatter(ob, [hv, wv], zeros16,
                                       mask=wv < _W_OUT)

    def start_in(q):
        p = q * _NW + wid
        sem = in_sems.at[q % 3]
        hi = pltpu.async_copy(
            idx_hbm.at[p // _C, p % _C], idx_bufs[q % 7], sem)
        hv = pltpu.async_copy(
            x_hbm.at[p // _C, p % _C], val_bufs[q % 4], sem)
        return hi, hv

    in_handles = {0: start_in(0), 1: start_in(1), 2: start_in(2)}
    out_handles = {}

    for q in range(_PLANES_PER_W):
        b = q % 3
        hi, hv = in_handles.pop(q)
        hi.wait()
        hv.wait()

        mask2 = lanes >= 2
        vecs = [(r, cb, None if cb == 0 else mask2)
                for r in range(_H_IN) for cb in (0, 14)]

        _D = 6
        if q >= 3:
            out_handles.pop(q - 3).wait()
            idx_old = idx_bufs[(q - 3) % 7]

            def load_i(k):
                r, cb, m = vecs[k]
                return idx_old[r, pl.ds(cb, 16)], m

            pend = [load_i(k) for k in range(_D)]
            for i in range(len(vecs)):
                if i + _D < len(vecs):
                    pend.append(load_i(i + _D))
                iv, m = pend.pop(0)
                hvv, wvv = _split_hw(iv)
                plsc.store_scatter(out_bufs[b], [hvv, wvv], zeros16, mask=m)

        if q + 3 < _PLANES_PER_W:
            in_handles[q + 3] = start_in(q + 3)

        idx_cur = idx_bufs[q % 7]
        val_cur = val_bufs[q % 4]

        def load_iv(k):
            r, cb, m = vecs[k]
            return idx_cur[r, pl.ds(cb, 16)], val_cur[r, pl.ds(cb, 16)], m

        pend = [load_iv(k) for k in range(_D)]
        for i in range(len(vecs)):
            if i + _D < len(vecs):
                pend.append(load_iv(i + _D))
            iv, vv, m = pend.pop(0)
            hvv, wvv = _split_hw(iv)
            plsc.store_scatter(out_bufs[b], [hvv, wvv], vv, mask=m)

        p = q * _NW + wid
        out_handles[q] = pltpu.async_copy(
            out_bufs[b], out_hbm.at[p // _C, :, p % _C, :], out_sems.at[b])

    for q in sorted(out_handles):
        out_handles.pop(q).wait()


@jax.jit
def kernel(x, indices):
    idx4 = indices.astype(jnp.int32)
    mesh = plsc.VectorSubcoreMesh(core_axis_name="c", subcore_axis_name="s")
    out = pl.kernel(
        _unpool_body,
        out_type=jax.ShapeDtypeStruct((_N, _H_OUT, _C, _W_OUT), jnp.float32),
        mesh=mesh,
        compiler_params=pltpu.CompilerParams(
            needs_layout_passes=False, use_tc_tiling_on_sc=True),
        scratch_types=[
            pltpu.VMEM((_H_IN, _W_IN), jnp.int32),
            pltpu.VMEM((_H_IN, _W_IN), jnp.int32),
            pltpu.VMEM((_H_IN, _W_IN), jnp.int32),
            pltpu.VMEM((_H_IN, _W_IN), jnp.int32),
            pltpu.VMEM((_H_IN, _W_IN), jnp.int32),
            pltpu.VMEM((_H_IN, _W_IN), jnp.int32),
            pltpu.VMEM((_H_IN, _W_IN), jnp.int32),
            pltpu.VMEM((_H_IN, _W_IN), jnp.float32),
            pltpu.VMEM((_H_IN, _W_IN), jnp.float32),
            pltpu.VMEM((_H_IN, _W_IN), jnp.float32),
            pltpu.VMEM((_H_IN, _W_IN), jnp.float32),
            pltpu.VMEM((_H_OUT, _W_OUT), jnp.float32),
            pltpu.VMEM((_H_OUT, _W_OUT), jnp.float32),
            pltpu.VMEM((_H_OUT, _W_OUT), jnp.float32),
            pltpu.SemaphoreType.DMA((3,)),
            pltpu.SemaphoreType.DMA((3,)),
        ],
    )(x, idx4)
    return jnp.transpose(out, (0, 2, 1, 3))


# speedup vs baseline: 1.0319x; 1.0319x over previous
import jax
import jax.numpy as jnp
from jax import lax
from jax.experimental import pallas as pl
from jax.experimental.pallas import tpu as pltpu, tpu_sc as plsc

_N, _C, _H_IN, _W_IN = 20, 16, 24, 30
_H_OUT, _W_OUT = 49, 61
_P = _N * _C
_S_IN = _H_IN * _W_IN
_NVEC = _S_IN // 16
_NW = 32
_PLANES_PER_W = _P // _NW
_DIV_MUL, _DIV_SHIFT = 4298, 18


def _split_hw(iv):
    hv = lax.shift_right_logical(iv * _DIV_MUL, _DIV_SHIFT)
    wv = iv - hv * _W_OUT
    return hv, wv


def _unpool_body(x_hbm, idx_hbm, out_hbm,
                 idx_v0, idx_v1, idx_v2, idx_v3, idx_v4,
                 val_v0, val_v1, val_v2, val_v3,
                 out_v0, out_v1, in_sems, out_sems):
    c = lax.axis_index("c")
    s = lax.axis_index("s")
    wid = s * 2 + c

    idx_bufs = [idx_v0, idx_v1, idx_v2, idx_v3, idx_v4]
    val_bufs = [val_v0, val_v1, val_v2, val_v3]
    out_bufs = [out_v0, out_v1]

    zeros16 = jnp.zeros((16,), jnp.float32)
    lanes = lax.iota(jnp.int32, 16)

    for ob in out_bufs:
        for r in range(_H_OUT):
            hv = jnp.full((16,), r, jnp.int32)
            for cb in range(0, _W_OUT, 16):
                wv = lanes + cb
                if cb + 16 <= _W_OUT:
                    plsc.store_scatter(ob, [hv, wv], zeros16)
                else:
                    plsc.store_scatter(ob, [hv, wv], zeros16,
                                       mask=wv < _W_OUT)

    def start_in(q):
        p = q * _NW + wid
        sem = in_sems.at[q % 3]
        hi = pltpu.async_copy(
            idx_hbm.at[p // _C, p % _C], idx_bufs[q % 5], sem)
        hv = pltpu.async_copy(
            x_hbm.at[p // _C, p % _C], val_bufs[q % 4], sem)
        return hi, hv

    in_handles = {0: start_in(0), 1: start_in(1), 2: start_in(2)}
    out_handles = {}

    for q in range(_PLANES_PER_W):
        b = q % 2
        hi, hv = in_handles.pop(q)
        hi.wait()
        hv.wait()

        mask2 = lanes >= 2
        vecs = [(r, cb, None if cb == 0 else mask2)
                for r in range(_H_IN) for cb in (0, 14)]

        _D = 6
        if q >= 2:
            out_handles.pop(q - 2).wait()
            idx_old = idx_bufs[(q - 2) % 5]

            def load_i(k):
                r, cb, m = vecs[k]
                return idx_old[r, pl.ds(cb, 16)], m

            pend = [load_i(k) for k in range(_D)]
            for i in range(len(vecs)):
                if i + _D < len(vecs):
                    pend.append(load_i(i + _D))
                iv, m = pend.pop(0)
                hvv, wvv = _split_hw(iv)
                plsc.store_scatter(out_bufs[b], [hvv, wvv], zeros16, mask=m)

        if q + 3 < _PLANES_PER_W:
            in_handles[q + 3] = start_in(q + 3)

        idx_cur = idx_bufs[q % 5]
        val_cur = val_bufs[q % 4]

        def load_iv(k):
            r, cb, m = vecs[k]
            return idx_cur[r, pl.ds(cb, 16)], val_cur[r, pl.ds(cb, 16)], m

        pend = [load_iv(k) for k in range(_D)]
        for i in range(len(vecs)):
            if i + _D < len(vecs):
                pend.append(load_iv(i + _D))
            iv, vv, m = pend.pop(0)
            hvv, wvv = _split_hw(iv)
            plsc.store_scatter(out_bufs[b], [hvv, wvv], vv, mask=m)

        p = q * _NW + wid
        out_handles[q] = pltpu.async_copy(
            out_bufs[b], out_hbm.at[p // _C, :, p % _C, :], out_sems.at[b])

    for q in sorted(out_handles):
        out_handles.pop(q).wait()


@jax.jit
def kernel(x, indices):
    idx4 = indices.astype(jnp.int32)
    mesh = plsc.VectorSubcoreMesh(core_axis_name="c", subcore_axis_name="s")
    out = pl.kernel(
        _unpool_body,
        out_type=jax.ShapeDtypeStruct((_N, _H_OUT, _C, _W_OUT), jnp.float32),
        mesh=mesh,
        compiler_params=pltpu.CompilerParams(
            needs_layout_passes=False, use_tc_tiling_on_sc=True),
        scratch_types=[
            pltpu.VMEM((_H_IN, _W_IN), jnp.int32),
            pltpu.VMEM((_H_IN, _W_IN), jnp.int32),
            pltpu.VMEM((_H_IN, _W_IN), jnp.int32),
            pltpu.VMEM((_H_IN, _W_IN), jnp.int32),
            pltpu.VMEM((_H_IN, _W_IN), jnp.int32),
            pltpu.VMEM((_H_IN, _W_IN), jnp.float32),
            pltpu.VMEM((_H_IN, _W_IN), jnp.float32),
            pltpu.VMEM((_H_IN, _W_IN), jnp.float32),
            pltpu.VMEM((_H_IN, _W_IN), jnp.float32),
            pltpu.VMEM((_H_OUT, _W_OUT), jnp.float32),
            pltpu.VMEM((_H_OUT, _W_OUT), jnp.float32),
            pltpu.SemaphoreType.DMA((3,)),
            pltpu.SemaphoreType.DMA((2,)),
        ],
    )(x, idx4)
    return jnp.transpose(out, (0, 2, 1, 3))


# speedup vs baseline: 1.0990x; 1.0650x over previous
import jax
import jax.numpy as jnp
from jax import lax
from jax.experimental import pallas as pl
from jax.experimental.pallas import tpu as pltpu, tpu_sc as plsc

_N, _C, _H_IN, _W_IN = 20, 16, 24, 30
_H_OUT, _W_OUT = 49, 61
_P = _N * _C
_NW = 32
_PLANES_PER_W = _P // _NW
_DIV_MUL, _DIV_SHIFT = 4298, 18
_D = 6


def _split_hw(iv):
    hv = lax.shift_right_logical(iv * _DIV_MUL, _DIV_SHIFT)
    wv = iv - hv * _W_OUT
    return hv, wv


def _unpool_body(x_hbm, idx_hbm, out_hbm,
                 idx_v0, idx_v1, idx_v2, idx_v3,
                 val_v0, val_v1, val_v2, val_v3,
                 out_v0, out_v1, in_sems, out_sems):
    c = lax.axis_index("c")
    s = lax.axis_index("s")
    wid = s * 2 + c

    idx_bufs = [idx_v0, idx_v1, idx_v2, idx_v3]
    val_bufs = [val_v0, val_v1, val_v2, val_v3]
    out_bufs = [out_v0, out_v1]

    zeros16 = jnp.zeros((16,), jnp.float32)
    lanes = lax.iota(jnp.int32, 16)
    mask2 = lanes >= 2
    vecs = [(r, cb, None if cb == 0 else mask2)
            for r in range(_H_IN) for cb in (0, 14)]

    for ob in out_bufs:
        for r in range(_H_OUT):
            hv = jnp.full((16,), r, jnp.int32)
            for cb in range(0, _W_OUT, 16):
                wv = lanes + cb
                if cb + 16 <= _W_OUT:
                    plsc.store_scatter(ob, [hv, wv], zeros16)
                else:
                    plsc.store_scatter(ob, [hv, wv], zeros16,
                                       mask=wv < _W_OUT)

    def in_copies(q, jm4):
        p = q * _NW + wid
        sem = in_sems.at[jm4 % 2]
        hi = pltpu.make_async_copy(
            idx_hbm.at[p // _C, p % _C], idx_bufs[jm4], sem)
        hv = pltpu.make_async_copy(
            x_hbm.at[p // _C, p % _C], val_bufs[jm4], sem)
        return hi, hv

    def out_copy(q, jm2):
        p = q * _NW + wid
        return pltpu.make_async_copy(
            out_bufs[jm2], out_hbm.at[p // _C, :, p % _C, :],
            out_sems.at[jm2])

    def wait_in(q, jm4):
        hi, hv = in_copies(q, jm4)
        hi.wait()
        hv.wait()

    def reset_plane(b, idx_old):
        def load_i(k):
            r, cb, m = vecs[k]
            return idx_old[r, pl.ds(cb, 16)], m

        pend = [load_i(k) for k in range(_D)]
        for i in range(len(vecs)):
            if i + _D < len(vecs):
                pend.append(load_i(i + _D))
            iv, m = pend.pop(0)
            hvv, wvv = _split_hw(iv)
            plsc.store_scatter(out_bufs[b], [hvv, wvv], zeros16, mask=m)

    def scatter_plane(b, idx_cur, val_cur):
        def load_iv(k):
            r, cb, m = vecs[k]
            return idx_cur[r, pl.ds(cb, 16)], val_cur[r, pl.ds(cb, 16)], m

        pend = [load_iv(k) for k in range(_D)]
        for i in range(len(vecs)):
            if i + _D < len(vecs):
                pend.append(load_iv(i + _D))
            iv, vv, m = pend.pop(0)
            hvv, wvv = _split_hw(iv)
            plsc.store_scatter(out_bufs[b], [hvv, wvv], vv, mask=m)

    for q in (0, 1):
        hi, hv = in_copies(q, q % 4)
        hi.start()
        hv.start()

    for q in (0, 1):
        wait_in(q, q % 4)
        hi, hv = in_copies(q + 2, (q + 2) % 4)
        hi.start()
        hv.start()
        scatter_plane(q % 2, idx_bufs[q % 4], val_bufs[q % 4])
        out_copy(q, q % 2).start()

    @pl.loop(0, 2)
    def _group(t):
        for j in range(4):
            q = 2 + t * 4 + j
            jm4 = (2 + j) % 4
            b = (2 + j) % 2
            wait_in(q, jm4)
            out_copy(q - 2, b).wait()
            reset_plane(b, idx_bufs[j])

            @pl.when(q + 2 < _PLANES_PER_W)
            def _():
                hi, hv = in_copies(q + 2, j)
                hi.start()
                hv.start()

            scatter_plane(b, idx_bufs[jm4], val_bufs[jm4])
            out_copy(q, b).start()

    for q in (8, 9):
        out_copy(q, q % 2).wait()


@jax.jit
def kernel(x, indices):
    idx4 = indices.astype(jnp.int32)
    mesh = plsc.VectorSubcoreMesh(core_axis_name="c", subcore_axis_name="s")
    out = pl.kernel(
        _unpool_body,
        out_type=jax.ShapeDtypeStruct((_N, _H_OUT, _C, _W_OUT), jnp.float32),
        mesh=mesh,
        compiler_params=pltpu.CompilerParams(
            needs_layout_passes=False, use_tc_tiling_on_sc=True),
        scratch_types=[
            pltpu.VMEM((_H_IN, _W_IN), jnp.int32),
            pltpu.VMEM((_H_IN, _W_IN), jnp.int32),
            pltpu.VMEM((_H_IN, _W_IN), jnp.int32),
            pltpu.VMEM((_H_IN, _W_IN), jnp.int32),
            pltpu.VMEM((_H_IN, _W_IN), jnp.float32),
            pltpu.VMEM((_H_IN, _W_IN), jnp.float32),
            pltpu.VMEM((_H_IN, _W_IN), jnp.float32),
            pltpu.VMEM((_H_IN, _W_IN), jnp.float32),
            pltpu.VMEM((_H_OUT, _W_OUT), jnp.float32),
            pltpu.VMEM((_H_OUT, _W_OUT), jnp.float32),
            pltpu.SemaphoreType.DMA((2,)),
            pltpu.SemaphoreType.DMA((2,)),
        ],
    )(x, idx4)
    return jnp.transpose(out, (0, 2, 1, 3))


# speedup vs baseline: 1.1706x; 1.0652x over previous
import jax
import jax.numpy as jnp
from jax import lax
from jax.experimental import pallas as pl
from jax.experimental.pallas import tpu as pltpu, tpu_sc as plsc

_N, _C, _H_IN, _W_IN = 20, 16, 24, 30
_H_OUT, _W_OUT = 49, 61
_P = _N * _C
_NW = 32
_PLANES_PER_W = _P // _NW
_DIV_MUL, _DIV_SHIFT = 4298, 18
_D = 6


def _split_hw(iv):
    hv = lax.shift_right_logical(iv * _DIV_MUL, _DIV_SHIFT)
    wv = iv - hv * _W_OUT
    return hv, wv


def _unpool_body(x_hbm, idx_hbm, out_hbm,
                 idx_v0, idx_v1, idx_v2, idx_v3,
                 val_v0, val_v1, val_v2, val_v3,
                 out_v0, out_v1, in_sems, out_sems):
    c = lax.axis_index("c")
    s = lax.axis_index("s")
    wid = s * 2 + c

    idx_bufs = [idx_v0, idx_v1, idx_v2, idx_v3]
    val_bufs = [val_v0, val_v1, val_v2, val_v3]
    out_bufs = [out_v0, out_v1]

    zeros16 = jnp.zeros((16,), jnp.float32)
    lanes = lax.iota(jnp.int32, 16)
    mask2 = lanes >= 2
    vecs = [(r, cb, None if cb == 0 else mask2)
            for r in range(_H_IN) for cb in (0, 14)]

    @pl.loop(0, _H_OUT)
    def _zero_row(r):
        hv = jnp.full((16,), r, jnp.int32)
        for ob in out_bufs:
            for cb in range(0, _W_OUT, 16):
                wv = lanes + cb
                if cb + 16 <= _W_OUT:
                    plsc.store_scatter(ob, [hv, wv], zeros16)
                else:
                    plsc.store_scatter(ob, [hv, wv], zeros16,
                                       mask=wv < _W_OUT)

    def in_copies(q, jm4):
        p = q * _NW + wid
        sem = in_sems.at[jm4 % 2]
        hi = pltpu.make_async_copy(
            idx_hbm.at[p // _C, p % _C], idx_bufs[jm4], sem)
        hv = pltpu.make_async_copy(
            x_hbm.at[p // _C, p % _C], val_bufs[jm4], sem)
        return hi, hv

    def out_copy(q, jm2):
        p = q * _NW + wid
        return pltpu.make_async_copy(
            out_bufs[jm2], out_hbm.at[p // _C, :, p % _C, :],
            out_sems.at[jm2])

    def wait_in(q, jm4):
        hi, hv = in_copies(q, jm4)
        hi.wait()
        hv.wait()

    def reset_plane(b, idx_old):
        def load_i(k):
            r, cb, m = vecs[k]
            return idx_old[r, pl.ds(cb, 16)], m

        pend = [load_i(k) for k in range(_D)]
        for i in range(len(vecs)):
            if i + _D < len(vecs):
                pend.append(load_i(i + _D))
            iv, m = pend.pop(0)
            hvv, wvv = _split_hw(iv)
            plsc.store_scatter(out_bufs[b], [hvv, wvv], zeros16, mask=m)

    def scatter_plane(b, idx_cur, val_cur):
        def load_iv(k):
            r, cb, m = vecs[k]
            return idx_cur[r, pl.ds(cb, 16)], val_cur[r, pl.ds(cb, 16)], m

        pend = [load_iv(k) for k in range(_D)]
        for i in range(len(vecs)):
            if i + _D < len(vecs):
                pend.append(load_iv(i + _D))
            iv, vv, m = pend.pop(0)
            hvv, wvv = _split_hw(iv)
            plsc.store_scatter(out_bufs[b], [hvv, wvv], vv, mask=m)

    for q in (0, 1):
        hi, hv = in_copies(q, q % 4)
        hi.start()
        hv.start()

    for q in (0, 1):
        wait_in(q, q % 4)
        hi, hv = in_copies(q + 2, (q + 2) % 4)
        hi.start()
        hv.start()
        scatter_plane(q % 2, idx_bufs[q % 4], val_bufs[q % 4])
        out_copy(q, q % 2).start()

    @pl.loop(0, 2)
    def _group(t):
        for j in range(4):
            q = 2 + t * 4 + j
            jm4 = (2 + j) % 4
            b = (2 + j) % 2
            wait_in(q, jm4)
            out_copy(q - 2, b).wait()
            reset_plane(b, idx_bufs[j])

            @pl.when(q + 2 < _PLANES_PER_W)
            def _():
                hi, hv = in_copies(q + 2, j)
                hi.start()
                hv.start()

            scatter_plane(b, idx_bufs[jm4], val_bufs[jm4])
            out_copy(q, b).start()

    for q in (8, 9):
        out_copy(q, q % 2).wait()


@jax.jit
def kernel(x, indices):
    idx4 = indices.astype(jnp.int32)
    mesh = plsc.VectorSubcoreMesh(core_axis_name="c", subcore_axis_name="s")
    out = pl.kernel(
        _unpool_body,
        out_type=jax.ShapeDtypeStruct((_N, _H_OUT, _C, _W_OUT), jnp.float32),
        mesh=mesh,
        compiler_params=pltpu.CompilerParams(
            needs_layout_passes=False, use_tc_tiling_on_sc=True),
        scratch_types=[
            pltpu.VMEM((_H_IN, _W_IN), jnp.int32),
            pltpu.VMEM((_H_IN, _W_IN), jnp.int32),
            pltpu.VMEM((_H_IN, _W_IN), jnp.int32),
            pltpu.VMEM((_H_IN, _W_IN), jnp.int32),
            pltpu.VMEM((_H_IN, _W_IN), jnp.float32),
            pltpu.VMEM((_H_IN, _W_IN), jnp.float32),
            pltpu.VMEM((_H_IN, _W_IN), jnp.float32),
            pltpu.VMEM((_H_IN, _W_IN), jnp.float32),
            pltpu.VMEM((_H_OUT, _W_OUT), jnp.float32),
            pltpu.VMEM((_H_OUT, _W_OUT), jnp.float32),
            pltpu.SemaphoreType.DMA((2,)),
            pltpu.SemaphoreType.DMA((2,)),
        ],
    )(x, idx4)
    return jnp.transpose(out, (0, 2, 1, 3))
